# baseline (speedup 1.0000x reference)
import jax
import jax.numpy as jnp
from jax import lax
from jax.experimental import pallas as pl
from jax.experimental.pallas import tpu as pltpu
from jax.experimental.pallas import tpu_sc as plsc

_B, _N, _D = 128, 64, 128
_NC, _NS, _L = 2, 16, 16
_NW = _NC * _NS
_BPW = _B // _NW
_NG = _D // _L
_INV = 1.0 / (_N - 1)

_mesh = plsc.VectorSubcoreMesh(core_axis_name="c", subcore_axis_name="s")


def _sc_encode_body(x_hbm, out_hbm, x_v, o_v):
    wid = lax.axis_index("s") * _NC + lax.axis_index("c")
    base = wid * _BPW
    pltpu.sync_copy(x_hbm.at[pl.ds(base, _BPW)], x_v)
    for bi in range(_BPW):
        zeros = tuple(jnp.zeros((_L,), jnp.float32) for _ in range(_NG))

        @plsc.parallel_loop(0, _N, unroll=4, carry=zeros)
        def accs(n, acc):
            return tuple(acc[g] + x_v[bi, n, pl.ds(_L * g, _L)] for g in range(_NG))

        @plsc.parallel_loop(0, _N, unroll=4)
        def _emit(n):
            for g in range(_NG):
                xv = x_v[bi, n, pl.ds(_L * g, _L)]
                o_v[bi, n, pl.ds(_L * g, _L)] = (accs[g] - xv) * _INV
                o_v[bi, n, pl.ds(_D + _L * g, _L)] = xv

    pltpu.sync_copy(o_v, out_hbm.at[pl.ds(base, _BPW)])


_sc_encode = pl.kernel(
    _sc_encode_body,
    out_type=jax.ShapeDtypeStruct((_B, _N, 2 * _D), jnp.float32),
    mesh=_mesh,
    scratch_types=[
        pltpu.VMEM((_BPW, _N, _D), jnp.float32),
        pltpu.VMEM((_BPW, _N, 2 * _D), jnp.float32),
    ],
)


def kernel(inputs, send_edges, recv_edges, edge2node_mat):
    return _sc_encode(inputs)


# speedup vs baseline: 5.4634x; 5.4634x over previous
import jax
import jax.numpy as jnp
from jax.experimental import pallas as pl


def _encode_block(x_ref, out_ref):
    x = x_ref[...]
    d = x.shape[2]
    inv = 1.0 / (x.shape[1] - 1)
    s = jnp.sum(x, axis=1, keepdims=True)
    out_ref[:, :, :d] = (s - x) * inv
    out_ref[:, :, d:] = x


def kernel(inputs, send_edges, recv_edges, edge2node_mat):
    b, n, d = inputs.shape
    bblk = 64
    return pl.pallas_call(
        _encode_block,
        grid=(b // bblk,),
        in_specs=[pl.BlockSpec((bblk, n, d), lambda i: (i, 0, 0))],
        out_specs=pl.BlockSpec((bblk, n, 2 * d), lambda i: (i, 0, 0)),
        out_shape=jax.ShapeDtypeStruct((b, n, 2 * d), inputs.dtype),
    )(inputs)


# speedup vs baseline: 5.4797x; 1.0030x over previous
import jax
import jax.numpy as jnp
from jax.experimental import pallas as pl


def _encode_block(x_ref, out_ref):
    x = x_ref[...]
    d = x.shape[2]
    inv = 1.0 / (x.shape[1] - 1)
    s = jnp.sum(x, axis=1, keepdims=True)
    out_ref[...] = jnp.concatenate([(s - x) * inv, x], axis=2)


def kernel(inputs, send_edges, recv_edges, edge2node_mat):
    b, n, d = inputs.shape
    bblk = 64
    return pl.pallas_call(
        _encode_block,
        grid=(b // bblk,),
        in_specs=[pl.BlockSpec((bblk, n, d), lambda i: (i, 0, 0))],
        out_specs=pl.BlockSpec((bblk, n, 2 * d), lambda i: (i, 0, 0)),
        out_shape=jax.ShapeDtypeStruct((b, n, 2 * d), inputs.dtype),
    )(inputs)
